# baseline (speedup 1.0000x reference)
import functools

import jax
import jax.numpy as jnp
from jax import lax
from jax.experimental import pallas as pl
from jax.experimental.pallas import tpu as pltpu
from jax.experimental.pallas import tpu_sc as plsc

_WIN = 4
_LANE = 128
_L = 16
_NSUB = 32
_NBIN = 256


def _temporal_kernel(x_ref, t_ref, out_ref, acc_ref):
    step = pl.program_id(0)
    x = x_ref[0]
    tg = t_ref[0]

    @pl.when(step == 0)
    def _():
        acc_ref[0] = 0.0

    mo = jnp.maximum(jnp.maximum(x[0], x[1]), jnp.maximum(x[2], x[3]))
    mt = jnp.maximum(jnp.maximum(tg[0], tg[1]), jnp.maximum(tg[2], tg[3]))
    dt = mo - mt
    acc_ref[0] = acc_ref[0] + jnp.sum(dt * dt)

    @pl.when(step == pl.num_programs(0) - 1)
    def _():
        out_ref[0, 0] = acc_ref[0]


def _temporal_sse(output, target):
    B, T, C, H, W = output.shape
    hw = H * W
    sub = hw // _LANE
    nw = T // _WIN
    xr = output.reshape(B * nw, _WIN, C, sub, _LANE)
    tr = target.reshape(B * nw, _WIN, C, sub, _LANE)
    spec = pl.BlockSpec((1, _WIN, C, sub, _LANE),
                        lambda r: (r, 0, 0, 0, 0))
    out = pl.pallas_call(
        _temporal_kernel,
        grid=(B * nw,),
        in_specs=[spec, spec],
        out_specs=pl.BlockSpec(memory_space=pltpu.SMEM),
        out_shape=jax.ShapeDtypeStruct((1, 1), jnp.float32),
        scratch_shapes=[pltpu.SMEM((1,), jnp.float32)],
    )(xr, tr)
    return out[0, 0]


def _sc_body(nrows, n, kk, rows_per, x_hbm, t_hbm, out_hbm,
             xv, tv, hist, p_ref, outv):
    cid = lax.axis_index("c")
    sid = lax.axis_index("s")
    wid = sid * 2 + cid

    iota = lax.iota(jnp.int32, _L)
    lane_base = iota * _NBIN
    ones = jnp.ones((_L,), jnp.int32)
    nchunk = n // _L
    i32min = jnp.int32(-2147483648)

    def zero_hist():
        def zbody(i, _):
            hist[pl.ds(i * _L, _L)] = jnp.zeros((_L,), jnp.int32)
            return 0
        lax.fori_loop(0, (_NBIN * _L) // _L, zbody, 0)

    def keys_of(v):
        bits = lax.bitcast_convert_type(v, jnp.int32)
        return jnp.where(bits < 0, bits ^ jnp.int32(0x7FFFFFFF), bits)

    def build_p():
        cum = jnp.int32(0)
        for c in range(_NBIN // _L):
            acc = jnp.zeros((_L,), jnp.int32)
            for j in range(_L):
                acc = acc + hist[pl.ds(j * _NBIN + c * _L, _L)]
            pc = plsc.cumsum(acc) + cum
            p_ref[pl.ds(c * _L, _L)] = pc
            cum = cum + jnp.sum(acc)
        return cum

    def find_cross(thresh):
        found = jnp.int32(0)
        b_star = jnp.int32(0)
        p_star = jnp.int32(0)
        prev = jnp.int32(0)
        for c in range(_NBIN // _L):
            pc = p_ref[pl.ds(c * _L, _L)]
            m = pc > thresh
            cand = jnp.where(m, 255 - (iota + c * _L), -1)
            mx = jnp.max(cand)
            bloc = 255 - mx
            pmin = -jnp.max(jnp.where(m, -pc, i32min))
            any_m = mx >= 0
            take = (found == 0) & any_m
            b_star = jnp.where(take, bloc, b_star)
            p_star = jnp.where(take, pmin, p_star)
            found = jnp.where(any_m, jnp.int32(1), found)
        for c in range(_NBIN // _L):
            pc = p_ref[pl.ds(c * _L, _L)]
            m2 = (iota + c * _L) < b_star
            prev = jnp.maximum(prev, jnp.max(jnp.where(m2, pc, 0)))
        return b_star, p_star, prev

    for r in range(rows_per):
        row = wid * rows_per + r
        if nrows % _NSUB != 0:
            row = jnp.minimum(row, nrows - 1)
        pltpu.sync_copy(x_hbm.at[row], xv)

        zero_hist()

        def p1_body(i, _):
            for j in range(8):
                v = xv[pl.ds((i * 8 + j) * _L, _L)]
                key = keys_of(v)
                bin1 = lax.shift_right_arithmetic(key, 24) + 128
                plsc.addupdate_scatter(hist, [lane_base + bin1], ones)
            return 0
        lax.fori_loop(0, nchunk // 8, p1_body, 0)
        build_p()
        b1, p1, _unused = find_cross(jnp.int32(n - kk))
        g8 = jnp.int32(n) - p1

        zero_hist()

        def p2_body(i, _):
            for j in range(8):
                v = xv[pl.ds((i * 8 + j) * _L, _L)]
                key = keys_of(v)
                bin1 = lax.shift_right_arithmetic(key, 24) + 128
                bin2 = lax.shift_right_arithmetic(key, 16) & 0xFF
                m = bin1 == b1
                plsc.addupdate_scatter(hist, [lane_base + bin2], ones,
                                       mask=m)
            return 0
        lax.fori_loop(0, nchunk // 8, p2_body, 0)
        e8 = build_p()
        b2, p2, prev2 = find_cross(g8 + e8 - jnp.int32(kk))
        g16 = g8 + (e8 - p2)
        e16 = p2 - prev2
        v16 = (b1 - 128) * 256 + b2

        pltpu.sync_copy(t_hbm.at[row], tv)

        def p3_body(i, carry):
            shi, sband = carry
            for j in range(8):
                sl = pl.ds((i * 8 + j) * _L, _L)
                v = xv[sl]
                tval = tv[sl]
                key16 = lax.shift_right_arithmetic(keys_of(v), 16)
                d = v - tval
                d2 = d * d
                shi = shi + jnp.where(key16 > v16, d2, 0.0)
                sband = sband + jnp.where(key16 == v16, d2, 0.0)
            return shi, sband
        z = jnp.zeros((_L,), jnp.float32)
        shi, sband = lax.fori_loop(0, nchunk // 8, p3_body, (z, z))
        s1 = jnp.sum(shi)
        s2 = jnp.sum(sband)
        vec = (jnp.where(iota == 0, s1, 0.0)
               + jnp.where(iota == 1, s2, 0.0)
               + jnp.where(iota == 2, g16.astype(jnp.float32), 0.0)
               + jnp.where(iota == 3, e16.astype(jnp.float32), 0.0))
        outv[r] = vec

    pltpu.sync_copy(outv, out_hbm.at[wid])


def _sc_spatial(x2d, t2d, kk):
    nrows, n = x2d.shape
    rows_per = (nrows + _NSUB - 1) // _NSUB
    mesh = plsc.VectorSubcoreMesh(core_axis_name="c", subcore_axis_name="s")
    body = functools.partial(_sc_body, nrows, n, kk, rows_per)
    f = pl.kernel(
        body,
        mesh=mesh,
        compiler_params=pltpu.CompilerParams(needs_layout_passes=False),
        out_type=jax.ShapeDtypeStruct((_NSUB, rows_per, _L), jnp.float32),
        scratch_types=[
            pltpu.VMEM((n,), jnp.float32),
            pltpu.VMEM((n,), jnp.float32),
            pltpu.VMEM((_NBIN * _L,), jnp.int32),
            pltpu.VMEM((_NBIN,), jnp.int32),
            pltpu.VMEM((rows_per, _L), jnp.float32),
        ],
    )
    return f(x2d, t2d)


def kernel(output, target):
    B, T, C, H, W = output.shape
    hw = H * W
    kk = hw // 10
    nrows = B * T * C
    xs = output.reshape(nrows, hw)
    ts = target.reshape(nrows, hw)
    sc_part = _sc_spatial(xs, ts, kk)
    time_sse = _temporal_sse(output, target)
    rows_per = sc_part.shape[1]
    p = sc_part.reshape(_NSUB * rows_per, _L)[:nrows]
    s_hi, s_band, g, e = p[:, 0], p[:, 1], p[:, 2], p[:, 3]
    spatial_sum = jnp.sum(s_hi + (kk - g) / e * s_band)
    tnorm = jnp.float32(B * C * hw * (T // _WIN))
    snorm = jnp.float32(nrows * kk)
    return time_sse / tnorm + spatial_sum / snorm
